# baseline (speedup 1.0000x reference)
import functools

import jax
import jax.numpy as jnp
from jax import lax
from jax.experimental import pallas as pl
from jax.experimental.pallas import tpu as pltpu
from jax.experimental.pallas import tpu_sc as plsc

N = 10000
E = 160000
STATIC = 256
SEQ_DIM = 16
T = 50
HID = 512
Z = 256
F = 64

NSUB = 16
E_PAD = 163840
EPT = E_PAD // NSUB
CH = 512
NCHUNK = EPT // CH
NROW = 10112
RPT = NROW // NSUB
LAST_L = N - (NSUB - 1) * RPT


def _sc_agg(table, srcs2, dst, zeros_f, zeros_d, ones_c, with_deg):
    mesh = plsc.VectorSubcoreMesh(core_axis_name="c", subcore_axis_name="s")
    out_type = [jax.ShapeDtypeStruct((4, N, F), jnp.float32)]
    scratch = [
        pltpu.VMEM((CH, F), jnp.float32),
        pltpu.VMEM((CH,), jnp.int32),
        pltpu.VMEM((CH,), jnp.int32),
        pltpu.VMEM_SHARED((NROW, F), jnp.float32),
    ]
    if with_deg:
        out_type.append(jax.ShapeDtypeStruct((N, 16), jnp.float32))
        scratch += [
            pltpu.VMEM((CH, 16), jnp.float32),
            pltpu.VMEM_SHARED((NROW, 16), jnp.float32),
        ]

    def body(table_h, srcs_h, dst_h, zf_h, zd_h, ones_h, agg_h, *rest):
        if with_deg:
            deg_h, rows_v, gidx_v, didx_v, acc_sh, ones_v, deg_sh = rest
        else:
            rows_v, gidx_v, didx_v, acc_sh = rest
        c = lax.axis_index("c")
        s = lax.axis_index("s")
        row0 = pl.multiple_of(s * RPT, 8)

        def zero_acc():
            pltpu.sync_copy(zf_h, acc_sh.at[pl.ds(row0, RPT)])

        def edge_pass(p):
            @pl.loop(0, NCHUNK)
            def _edges(k):
                base = pl.multiple_of(s * EPT + k * CH, 128)
                pltpu.sync_copy(srcs_h.at[c].at[pl.ds(base, CH)], gidx_v)
                pltpu.sync_copy(dst_h.at[pl.ds(base, CH)], didx_v)
                if p:
                    @pl.loop(0, CH // 16)
                    def _adj(i):
                        sl = pl.ds(i * 16, 16)
                        gidx_v[sl] = gidx_v[sl] + jnp.int32(p * N)
                pltpu.sync_copy(table_h.at[gidx_v], rows_v)
                pltpu.sync_copy(rows_v, acc_sh.at[didx_v], add=True)
                if with_deg and p == 0:
                    pltpu.sync_copy(ones_v, deg_sh.at[didx_v], add=True)

        def writeback(p, length):
            pltpu.sync_copy(acc_sh.at[pl.ds(row0, length)],
                            agg_h.at[2 * c + p].at[pl.ds(row0, length)])
            if with_deg and p == 0:
                @pl.when(c == 0)
                def _():
                    pltpu.sync_copy(deg_sh.at[pl.ds(row0, length)],
                                    deg_h.at[pl.ds(row0, length)])

        def writeback_sized(p):
            @pl.when(s < NSUB - 1)
            def _():
                writeback(p, RPT)

            @pl.when(s == NSUB - 1)
            def _():
                writeback(p, LAST_L)

        zero_acc()
        if with_deg:
            pltpu.sync_copy(zd_h, deg_sh.at[pl.ds(row0, RPT)])
            pltpu.sync_copy(ones_h, ones_v)
        plsc.subcore_barrier()
        edge_pass(0)
        plsc.subcore_barrier()
        writeback_sized(0)
        zero_acc()
        plsc.subcore_barrier()
        edge_pass(1)
        plsc.subcore_barrier()
        writeback_sized(1)

    k = pl.kernel(body, mesh=mesh, out_type=out_type, scratch_types=scratch,
                  compiler_params=pltpu.CompilerParams(
                      use_tc_tiling_on_sc=False))
    return k(table, srcs2, dst, zeros_f, zeros_d, ones_c)


def _gru_dir(x_seq_t, wihT, whhT, bih, bhh, reverse):
    B = 400
    nb = N // B

    def body(x_ref, wih_ref, whh_ref, bih_ref, bhh_ref, o_ref):
        def step(t, h):
            tt = (T - 1 - t) if reverse else t
            xt = x_ref[tt].astype(jnp.bfloat16)
            gi = jnp.dot(xt, wih_ref[...],
                         preferred_element_type=jnp.float32) + bih_ref[...]
            gh = jnp.dot(h.astype(jnp.bfloat16), whh_ref[...],
                         preferred_element_type=jnp.float32) + bhh_ref[...]
            r = jax.nn.sigmoid(gi[:, :HID] + gh[:, :HID])
            z = jax.nn.sigmoid(gi[:, HID:2 * HID] + gh[:, HID:2 * HID])
            n = jnp.tanh(gi[:, 2 * HID:] + r * gh[:, 2 * HID:])
            return (1.0 - z) * n + z * h

        o_ref[...] = lax.fori_loop(0, T, step, jnp.zeros((B, HID), jnp.float32))

    return pl.pallas_call(
        body,
        grid=(nb,),
        in_specs=[
            pl.BlockSpec((T, B, SEQ_DIM), lambda i: (0, i, 0)),
            pl.BlockSpec((SEQ_DIM, 3 * HID), lambda i: (0, 0)),
            pl.BlockSpec((HID, 3 * HID), lambda i: (0, 0)),
            pl.BlockSpec((1, 3 * HID), lambda i: (0, 0)),
            pl.BlockSpec((1, 3 * HID), lambda i: (0, 0)),
        ],
        out_specs=pl.BlockSpec((B, HID), lambda i: (i, 0)),
        out_shape=jax.ShapeDtypeStruct((N, HID), jnp.float32),
    )(x_seq_t, wihT, whhT, bih, bhh)


def _mat1(mean1_num, deg, x_static, wl1T, bl1, wr1T, wl2T, wr2T, bl2):
    B = 2000
    nb = N // B

    def body(agg_ref, deg_ref, x_ref, wl1_ref, bl1_ref, wr1_ref, wl2_ref,
             wr2_ref, bl2_ref, y2_ref, hr2_ref):
        inv = 1.0 / jnp.maximum(deg_ref[:, 0:1], 1.0)
        mean1 = agg_ref[...] * inv
        h = jnp.maximum(
            jnp.dot(mean1, wl1_ref[...], preferred_element_type=jnp.float32)
            + bl1_ref[...]
            + jnp.dot(x_ref[...], wr1_ref[...],
                      preferred_element_type=jnp.float32), 0.0)
        y2_ref[...] = jnp.dot(h, wl2_ref[...],
                              preferred_element_type=jnp.float32)
        hr2_ref[...] = jnp.dot(h, wr2_ref[...],
                               preferred_element_type=jnp.float32) + bl2_ref[...]

    return pl.pallas_call(
        body,
        grid=(nb,),
        in_specs=[
            pl.BlockSpec((B, STATIC), lambda i: (i, 0)),
            pl.BlockSpec((B, 16), lambda i: (i, 0)),
            pl.BlockSpec((B, STATIC), lambda i: (i, 0)),
            pl.BlockSpec((STATIC, HID), lambda i: (0, 0)),
            pl.BlockSpec((1, HID), lambda i: (0, 0)),
            pl.BlockSpec((STATIC, HID), lambda i: (0, 0)),
            pl.BlockSpec((HID, Z), lambda i: (0, 0)),
            pl.BlockSpec((HID, Z), lambda i: (0, 0)),
            pl.BlockSpec((1, Z), lambda i: (0, 0)),
        ],
        out_specs=[
            pl.BlockSpec((B, Z), lambda i: (i, 0)),
            pl.BlockSpec((B, Z), lambda i: (i, 0)),
        ],
        out_shape=[
            jax.ShapeDtypeStruct((N, Z), jnp.float32),
            jax.ShapeDtypeStruct((N, Z), jnp.float32),
        ],
    )(mean1_num, deg, x_static, wl1T, bl1, wr1T, wl2T, wr2T, bl2)


def _final(agg2, deg, hr2, hf, hb, wfc1T, wfc2T, bfc, wgT, wmT, bfu):
    B = 2000
    nb = N // B

    def body(agg_ref, deg_ref, hr2_ref, hf_ref, hb_ref, wfc1_ref, wfc2_ref,
             bfc_ref, wg_ref, wm_ref, bfu_ref, o_ref):
        inv = 1.0 / jnp.maximum(deg_ref[:, 0:1], 1.0)
        zg = agg_ref[...] * inv + hr2_ref[...]
        zm = (jnp.dot(hf_ref[...], wfc1_ref[...],
                      preferred_element_type=jnp.float32)
              + jnp.dot(hb_ref[...], wfc2_ref[...],
                        preferred_element_type=jnp.float32) + bfc_ref[...])
        o_ref[...] = (jnp.dot(zg, wg_ref[...],
                              preferred_element_type=jnp.float32)
                      + jnp.dot(zm, wm_ref[...],
                                preferred_element_type=jnp.float32)
                      + bfu_ref[...])

    return pl.pallas_call(
        body,
        grid=(nb,),
        in_specs=[
            pl.BlockSpec((B, Z), lambda i: (i, 0)),
            pl.BlockSpec((B, 16), lambda i: (i, 0)),
            pl.BlockSpec((B, Z), lambda i: (i, 0)),
            pl.BlockSpec((B, HID), lambda i: (i, 0)),
            pl.BlockSpec((B, HID), lambda i: (i, 0)),
            pl.BlockSpec((HID, Z), lambda i: (0, 0)),
            pl.BlockSpec((HID, Z), lambda i: (0, 0)),
            pl.BlockSpec((1, Z), lambda i: (0, 0)),
            pl.BlockSpec((Z, Z), lambda i: (0, 0)),
            pl.BlockSpec((Z, Z), lambda i: (0, 0)),
            pl.BlockSpec((1, Z), lambda i: (0, 0)),
        ],
        out_specs=pl.BlockSpec((B, Z), lambda i: (i, 0)),
        out_shape=jax.ShapeDtypeStruct((N, Z), jnp.float32),
    )(agg2, deg, hr2, hf, hb, wfc1T, wfc2T, bfc, wgT, wmT, bfu)


def _to_table(x):
    return x.reshape(N, 4, F).transpose(1, 0, 2).reshape(4 * N, F)


def _from_quarters(a):
    return a.transpose(1, 0, 2).reshape(N, 4 * F)


def kernel(x_static, edge_index, x_seq, Wl1, bl1, Wr1, Wl2, bl2, Wr2,
           Wih_f, Whh_f, bih_f, bhh_f, Wih_b, Whh_b, bih_b, bhh_b,
           Wfc, bfc, Wfu, bfu):
    pad = E_PAD - E
    src = jnp.concatenate([edge_index[0], jnp.zeros((pad,), jnp.int32)])
    dst = jnp.concatenate([edge_index[1], jnp.full((pad,), N, jnp.int32)])
    srcs2 = jnp.stack([src, src + 2 * N])
    zeros_f = jnp.zeros((RPT, F), jnp.float32)
    zeros_d = jnp.zeros((RPT, 16), jnp.float32)
    ones_c = jnp.ones((CH, 16), jnp.float32)

    agg1, deg = _sc_agg(_to_table(x_static), srcs2, dst,
                        zeros_f, zeros_d, ones_c, True)

    x_seq_t = jnp.swapaxes(x_seq, 0, 1)
    hf = _gru_dir(x_seq_t, Wih_f.T.astype(jnp.bfloat16),
                  Whh_f.T.astype(jnp.bfloat16),
                  bih_f.reshape(1, -1), bhh_f.reshape(1, -1), False)

    y2, hr2 = _mat1(_from_quarters(agg1), deg, x_static, Wl1.T,
                    bl1.reshape(1, -1), Wr1.T, Wl2.T, Wr2.T,
                    bl2.reshape(1, -1))

    (agg2,) = _sc_agg(_to_table(y2), srcs2, dst,
                      zeros_f, zeros_d, ones_c, False)

    hb = _gru_dir(x_seq_t, Wih_b.T.astype(jnp.bfloat16),
                  Whh_b.T.astype(jnp.bfloat16),
                  bih_b.reshape(1, -1), bhh_b.reshape(1, -1), True)

    return _final(_from_quarters(agg2), deg, hr2, hf, hb,
                  Wfc[:, :HID].T, Wfc[:, HID:].T, bfc.reshape(1, -1),
                  Wfu[:, :Z].T, Wfu[:, Z:].T, bfu.reshape(1, -1))


# speedup vs baseline: 1.0794x; 1.0794x over previous
import functools

import jax
import jax.numpy as jnp
from jax import lax
from jax.experimental import pallas as pl
from jax.experimental.pallas import tpu as pltpu
from jax.experimental.pallas import tpu_sc as plsc

N = 10000
E = 160000
STATIC = 256
SEQ_DIM = 16
T = 50
HID = 512
Z = 256
F = 64

NSUB = 16
E_PAD = 163840
EPT = E_PAD // NSUB
CH = 512
NCHUNK = EPT // CH
NROW = 10112
RPT = NROW // NSUB
LAST_L = N - (NSUB - 1) * RPT


def _sc_agg(table, srcs2, dst, zeros_f, zeros_d, ones_c, with_deg):
    mesh = plsc.VectorSubcoreMesh(core_axis_name="c", subcore_axis_name="s")
    out_type = [jax.ShapeDtypeStruct((4, N, F), jnp.float32)]
    scratch = [
        pltpu.VMEM((CH, F), jnp.float32),
        pltpu.VMEM((CH,), jnp.int32),
        pltpu.VMEM((CH,), jnp.int32),
        pltpu.VMEM_SHARED((NROW, F), jnp.float32),
    ]
    if with_deg:
        out_type.append(jax.ShapeDtypeStruct((N, 16), jnp.float32))
        scratch += [
            pltpu.VMEM((CH, 16), jnp.float32),
            pltpu.VMEM_SHARED((NROW, 16), jnp.float32),
        ]

    def body(table_h, srcs_h, dst_h, zf_h, zd_h, ones_h, agg_h, *rest):
        if with_deg:
            deg_h, rows_v, gidx_v, didx_v, acc_sh, ones_v, deg_sh = rest
        else:
            rows_v, gidx_v, didx_v, acc_sh = rest
        c = lax.axis_index("c")
        s = lax.axis_index("s")
        row0 = pl.multiple_of(s * RPT, 8)

        def zero_acc():
            pltpu.sync_copy(zf_h, acc_sh.at[pl.ds(row0, RPT)])

        def edge_pass(p):
            @pl.loop(0, NCHUNK)
            def _edges(k):
                base = pl.multiple_of(s * EPT + k * CH, 128)
                pltpu.sync_copy(srcs_h.at[c].at[pl.ds(base, CH)], gidx_v)
                pltpu.sync_copy(dst_h.at[pl.ds(base, CH)], didx_v)
                if p:
                    @pl.loop(0, CH // 16)
                    def _adj(i):
                        sl = pl.ds(i * 16, 16)
                        gidx_v[sl] = gidx_v[sl] + jnp.int32(p * N)
                pltpu.sync_copy(table_h.at[gidx_v], rows_v)
                pltpu.sync_copy(rows_v, acc_sh.at[didx_v], add=True)
                if with_deg and p == 0:
                    pltpu.sync_copy(ones_v, deg_sh.at[didx_v], add=True)

        def writeback(p, length):
            pltpu.sync_copy(acc_sh.at[pl.ds(row0, length)],
                            agg_h.at[2 * c + p].at[pl.ds(row0, length)])
            if with_deg and p == 0:
                @pl.when(c == 0)
                def _():
                    pltpu.sync_copy(deg_sh.at[pl.ds(row0, length)],
                                    deg_h.at[pl.ds(row0, length)])

        def writeback_sized(p):
            @pl.when(s < NSUB - 1)
            def _():
                writeback(p, RPT)

            @pl.when(s == NSUB - 1)
            def _():
                writeback(p, LAST_L)

        zero_acc()
        if with_deg:
            pltpu.sync_copy(zd_h, deg_sh.at[pl.ds(row0, RPT)])
            pltpu.sync_copy(ones_h, ones_v)
        plsc.subcore_barrier()
        edge_pass(0)
        plsc.subcore_barrier()
        writeback_sized(0)
        zero_acc()
        plsc.subcore_barrier()
        edge_pass(1)
        plsc.subcore_barrier()
        writeback_sized(1)

    k = pl.kernel(body, mesh=mesh, out_type=out_type, scratch_types=scratch,
                  compiler_params=pltpu.CompilerParams(
                      use_tc_tiling_on_sc=False))
    return k(table, srcs2, dst, zeros_f, zeros_d, ones_c)


def _gru_dir(x_seq_t, wrz, win, whn, brz, bni, bnh, reverse):
    B = 1000
    nb = N // B

    def body(x_ref, wrz_ref, win_ref, whn_ref, brz_ref, bni_ref, bnh_ref,
             o_ref):
        def step(t, h):
            tt = (T - 1 - t) if reverse else t
            xt = x_ref[tt]
            xh = jnp.concatenate([xt, h], axis=1)
            rz = jax.nn.sigmoid(
                jnp.dot(xh, wrz_ref[...],
                        preferred_element_type=jnp.float32) + brz_ref[...])
            n = jnp.tanh(
                jnp.dot(xt, win_ref[...],
                        preferred_element_type=jnp.float32) + bni_ref[...]
                + rz[:, :HID]
                * (jnp.dot(h, whn_ref[...],
                           preferred_element_type=jnp.float32) + bnh_ref[...]))
            z = rz[:, HID:]
            return ((1.0 - z) * n + z * h).astype(jnp.bfloat16)

        h_fin = lax.fori_loop(0, T, step, jnp.zeros((B, HID), jnp.bfloat16))
        o_ref[...] = h_fin.astype(jnp.float32)

    return pl.pallas_call(
        body,
        grid=(nb,),
        in_specs=[
            pl.BlockSpec((T, B, SEQ_DIM), lambda i: (0, i, 0)),
            pl.BlockSpec((SEQ_DIM + HID, 2 * HID), lambda i: (0, 0)),
            pl.BlockSpec((SEQ_DIM, HID), lambda i: (0, 0)),
            pl.BlockSpec((HID, HID), lambda i: (0, 0)),
            pl.BlockSpec((1, 2 * HID), lambda i: (0, 0)),
            pl.BlockSpec((1, HID), lambda i: (0, 0)),
            pl.BlockSpec((1, HID), lambda i: (0, 0)),
        ],
        out_specs=pl.BlockSpec((B, HID), lambda i: (i, 0)),
        out_shape=jax.ShapeDtypeStruct((N, HID), jnp.float32),
    )(x_seq_t, wrz, win, whn, brz, bni, bnh)


def _gru_weights(Wih, Whh, bih, bhh):
    wihT = Wih.T.astype(jnp.bfloat16)
    whhT = Whh.T.astype(jnp.bfloat16)
    wrz = jnp.concatenate([wihT[:, :2 * HID], whhT[:, :2 * HID]], axis=0)
    win = wihT[:, 2 * HID:]
    whn = whhT[:, 2 * HID:]
    brz = (bih[:2 * HID] + bhh[:2 * HID]).reshape(1, -1)
    bni = bih[2 * HID:].reshape(1, -1)
    bnh = bhh[2 * HID:].reshape(1, -1)
    return wrz, win, whn, brz, bni, bnh


def _mat1(mean1_num, deg, x_static, wl1T, bl1, wr1T, wl2T, wr2T, bl2):
    B = 2000
    nb = N // B

    def body(agg_ref, deg_ref, x_ref, wl1_ref, bl1_ref, wr1_ref, wl2_ref,
             wr2_ref, bl2_ref, y2_ref, hr2_ref):
        inv = 1.0 / jnp.maximum(deg_ref[:, 0:1], 1.0)
        mean1 = agg_ref[...] * inv
        h = jnp.maximum(
            jnp.dot(mean1, wl1_ref[...], preferred_element_type=jnp.float32)
            + bl1_ref[...]
            + jnp.dot(x_ref[...], wr1_ref[...],
                      preferred_element_type=jnp.float32), 0.0)
        y2_ref[...] = jnp.dot(h, wl2_ref[...],
                              preferred_element_type=jnp.float32)
        hr2_ref[...] = jnp.dot(h, wr2_ref[...],
                               preferred_element_type=jnp.float32) + bl2_ref[...]

    return pl.pallas_call(
        body,
        grid=(nb,),
        in_specs=[
            pl.BlockSpec((B, STATIC), lambda i: (i, 0)),
            pl.BlockSpec((B, 16), lambda i: (i, 0)),
            pl.BlockSpec((B, STATIC), lambda i: (i, 0)),
            pl.BlockSpec((STATIC, HID), lambda i: (0, 0)),
            pl.BlockSpec((1, HID), lambda i: (0, 0)),
            pl.BlockSpec((STATIC, HID), lambda i: (0, 0)),
            pl.BlockSpec((HID, Z), lambda i: (0, 0)),
            pl.BlockSpec((HID, Z), lambda i: (0, 0)),
            pl.BlockSpec((1, Z), lambda i: (0, 0)),
        ],
        out_specs=[
            pl.BlockSpec((B, Z), lambda i: (i, 0)),
            pl.BlockSpec((B, Z), lambda i: (i, 0)),
        ],
        out_shape=[
            jax.ShapeDtypeStruct((N, Z), jnp.float32),
            jax.ShapeDtypeStruct((N, Z), jnp.float32),
        ],
    )(mean1_num, deg, x_static, wl1T, bl1, wr1T, wl2T, wr2T, bl2)


def _final(agg2, deg, hr2, hf, hb, wfc1T, wfc2T, bfc, wgT, wmT, bfu):
    B = 2000
    nb = N // B

    def body(agg_ref, deg_ref, hr2_ref, hf_ref, hb_ref, wfc1_ref, wfc2_ref,
             bfc_ref, wg_ref, wm_ref, bfu_ref, o_ref):
        inv = 1.0 / jnp.maximum(deg_ref[:, 0:1], 1.0)
        zg = agg_ref[...] * inv + hr2_ref[...]
        zm = (jnp.dot(hf_ref[...], wfc1_ref[...],
                      preferred_element_type=jnp.float32)
              + jnp.dot(hb_ref[...], wfc2_ref[...],
                        preferred_element_type=jnp.float32) + bfc_ref[...])
        o_ref[...] = (jnp.dot(zg, wg_ref[...],
                              preferred_element_type=jnp.float32)
                      + jnp.dot(zm, wm_ref[...],
                                preferred_element_type=jnp.float32)
                      + bfu_ref[...])

    return pl.pallas_call(
        body,
        grid=(nb,),
        in_specs=[
            pl.BlockSpec((B, Z), lambda i: (i, 0)),
            pl.BlockSpec((B, 16), lambda i: (i, 0)),
            pl.BlockSpec((B, Z), lambda i: (i, 0)),
            pl.BlockSpec((B, HID), lambda i: (i, 0)),
            pl.BlockSpec((B, HID), lambda i: (i, 0)),
            pl.BlockSpec((HID, Z), lambda i: (0, 0)),
            pl.BlockSpec((HID, Z), lambda i: (0, 0)),
            pl.BlockSpec((1, Z), lambda i: (0, 0)),
            pl.BlockSpec((Z, Z), lambda i: (0, 0)),
            pl.BlockSpec((Z, Z), lambda i: (0, 0)),
            pl.BlockSpec((1, Z), lambda i: (0, 0)),
        ],
        out_specs=pl.BlockSpec((B, Z), lambda i: (i, 0)),
        out_shape=jax.ShapeDtypeStruct((N, Z), jnp.float32),
    )(agg2, deg, hr2, hf, hb, wfc1T, wfc2T, bfc, wgT, wmT, bfu)


def _to_table(x):
    return x.reshape(N, 4, F).transpose(1, 0, 2).reshape(4 * N, F)


def _from_quarters(a):
    return a.transpose(1, 0, 2).reshape(N, 4 * F)


def kernel(x_static, edge_index, x_seq, Wl1, bl1, Wr1, Wl2, bl2, Wr2,
           Wih_f, Whh_f, bih_f, bhh_f, Wih_b, Whh_b, bih_b, bhh_b,
           Wfc, bfc, Wfu, bfu):
    pad = E_PAD - E
    src = jnp.concatenate([edge_index[0], jnp.zeros((pad,), jnp.int32)])
    dst = jnp.concatenate([edge_index[1], jnp.full((pad,), N, jnp.int32)])
    srcs2 = jnp.stack([src, src + 2 * N])
    zeros_f = jnp.zeros((RPT, F), jnp.float32)
    zeros_d = jnp.zeros((RPT, 16), jnp.float32)
    ones_c = jnp.ones((CH, 16), jnp.float32)

    agg1, deg = _sc_agg(_to_table(x_static), srcs2, dst,
                        zeros_f, zeros_d, ones_c, True)

    x_seq_t = jnp.swapaxes(x_seq, 0, 1).astype(jnp.bfloat16)
    hf = _gru_dir(x_seq_t, *_gru_weights(Wih_f, Whh_f, bih_f, bhh_f), False)

    y2, hr2 = _mat1(_from_quarters(agg1), deg, x_static, Wl1.T,
                    bl1.reshape(1, -1), Wr1.T, Wl2.T, Wr2.T,
                    bl2.reshape(1, -1))

    (agg2,) = _sc_agg(_to_table(y2), srcs2, dst,
                      zeros_f, zeros_d, ones_c, False)

    hb = _gru_dir(x_seq_t, *_gru_weights(Wih_b, Whh_b, bih_b, bhh_b), True)

    return _final(_from_quarters(agg2), deg, hr2, hf, hb,
                  Wfc[:, :HID].T, Wfc[:, HID:].T, bfc.reshape(1, -1),
                  Wfu[:, :Z].T, Wfu[:, Z:].T, bfu.reshape(1, -1))


# speedup vs baseline: 1.1380x; 1.0543x over previous
import functools

import jax
import jax.numpy as jnp
from jax import lax
from jax.experimental import pallas as pl
from jax.experimental.pallas import tpu as pltpu
from jax.experimental.pallas import tpu_sc as plsc

N = 10000
E = 160000
STATIC = 256
SEQ_DIM = 16
T = 50
HID = 512
Z = 256
F = 64

NSUB = 16
E_PAD = 163840
EPT = E_PAD // NSUB
CH = 512
NCHUNK = EPT // CH
NROW = 10112
RPT = NROW // NSUB
LAST_L = N - (NSUB - 1) * RPT


def _sc_agg(table, srcs2, dst, zeros_f, zeros_d, ones_c, with_deg):
    mesh = plsc.VectorSubcoreMesh(core_axis_name="c", subcore_axis_name="s")
    out_type = [jax.ShapeDtypeStruct((4, N, F), jnp.float32)]
    scratch = [
        pltpu.VMEM((CH, F), jnp.float32),
        pltpu.VMEM((CH, F), jnp.float32),
        pltpu.VMEM((CH,), jnp.int32),
        pltpu.VMEM((CH,), jnp.int32),
        pltpu.VMEM((CH,), jnp.int32),
        pltpu.VMEM((CH,), jnp.int32),
        pltpu.SemaphoreType.DMA,
        pltpu.SemaphoreType.DMA,
        pltpu.VMEM_SHARED((NROW, F), jnp.float32),
    ]
    if with_deg:
        out_type.append(jax.ShapeDtypeStruct((N, 16), jnp.float32))
        scratch += [
            pltpu.VMEM((CH, 16), jnp.float32),
            pltpu.VMEM_SHARED((NROW, 16), jnp.float32),
        ]

    def body(table_h, srcs_h, dst_h, zf_h, zd_h, ones_h, agg_h, *rest):
        if with_deg:
            (deg_h, rows_a, rows_b, gidx_a, gidx_b, didx_a, didx_b,
             sem_a, sem_b, acc_sh, ones_v, deg_sh) = rest
        else:
            (rows_a, rows_b, gidx_a, gidx_b, didx_a, didx_b,
             sem_a, sem_b, acc_sh) = rest
        c = lax.axis_index("c")
        s = lax.axis_index("s")
        row0 = pl.multiple_of(s * RPT, 8)

        def zero_acc():
            pltpu.sync_copy(zf_h, acc_sh.at[pl.ds(row0, RPT)])

        def edge_pass(p):
            def load_idx(k, gidx_v, didx_v):
                base = pl.multiple_of(s * EPT + k * CH, 128)
                pltpu.sync_copy(srcs_h.at[c].at[pl.ds(base, CH)], gidx_v)
                pltpu.sync_copy(dst_h.at[pl.ds(base, CH)], didx_v)
                if p:
                    @pl.loop(0, CH // 16)
                    def _adj(i):
                        sl = pl.ds(i * 16, 16)
                        gidx_v[sl] = gidx_v[sl] + jnp.int32(p * N)

            def start_gather(gidx_v, rows_v, sem):
                pltpu.async_copy(table_h.at[gidx_v], rows_v, sem)

            def finish(rows_v, gidx_v, didx_v, sem):
                pltpu.make_async_copy(table_h.at[gidx_v], rows_v, sem).wait()
                pltpu.sync_copy(rows_v, acc_sh.at[didx_v], add=True)
                if with_deg and p == 0:
                    pltpu.sync_copy(ones_v, deg_sh.at[didx_v], add=True)

            load_idx(0, gidx_a, didx_a)
            start_gather(gidx_a, rows_a, sem_a)

            @pl.loop(0, NCHUNK, step=2)
            def _edges(k):
                load_idx(k + 1, gidx_b, didx_b)
                start_gather(gidx_b, rows_b, sem_b)
                finish(rows_a, gidx_a, didx_a, sem_a)

                @pl.when(k + 2 < NCHUNK)
                def _():
                    load_idx(k + 2, gidx_a, didx_a)
                    start_gather(gidx_a, rows_a, sem_a)

                finish(rows_b, gidx_b, didx_b, sem_b)

        def writeback(p, length):
            pltpu.sync_copy(acc_sh.at[pl.ds(row0, length)],
                            agg_h.at[2 * c + p].at[pl.ds(row0, length)])
            if with_deg and p == 0:
                @pl.when(c == 0)
                def _():
                    pltpu.sync_copy(deg_sh.at[pl.ds(row0, length)],
                                    deg_h.at[pl.ds(row0, length)])

        def writeback_sized(p):
            @pl.when(s < NSUB - 1)
            def _():
                writeback(p, RPT)

            @pl.when(s == NSUB - 1)
            def _():
                writeback(p, LAST_L)

        zero_acc()
        if with_deg:
            pltpu.sync_copy(zd_h, deg_sh.at[pl.ds(row0, RPT)])
            pltpu.sync_copy(ones_h, ones_v)
        plsc.subcore_barrier()
        edge_pass(0)
        plsc.subcore_barrier()
        writeback_sized(0)
        zero_acc()
        plsc.subcore_barrier()
        edge_pass(1)
        plsc.subcore_barrier()
        writeback_sized(1)

    k = pl.kernel(body, mesh=mesh, out_type=out_type, scratch_types=scratch,
                  compiler_params=pltpu.CompilerParams(
                      use_tc_tiling_on_sc=False))
    return k(table, srcs2, dst, zeros_f, zeros_d, ones_c)


def _gru_dir(x_seq_t, wrz, win, whn, brz, bni, bnh, reverse):
    B = 1000
    nb = N // B

    def body(x_ref, wrz_ref, win_ref, whn_ref, brz_ref, bni_ref, bnh_ref,
             o_ref):
        def step(t, h):
            tt = (T - 1 - t) if reverse else t
            xt = x_ref[tt]
            xh = jnp.concatenate([xt, h], axis=1)
            rz = jax.nn.sigmoid(
                jnp.dot(xh, wrz_ref[...],
                        preferred_element_type=jnp.float32)
                + brz_ref[...]).astype(jnp.bfloat16)
            gn = (jnp.dot(xt, win_ref[...],
                          preferred_element_type=jnp.float32)
                  + bni_ref[...]).astype(jnp.bfloat16)
            gh = (jnp.dot(h, whn_ref[...],
                          preferred_element_type=jnp.float32)
                  + bnh_ref[...]).astype(jnp.bfloat16)
            n = jnp.tanh(gn + rz[:, :HID] * gh)
            z = rz[:, HID:]
            return (1.0 - z) * n + z * h

        h_fin = lax.fori_loop(0, T, step, jnp.zeros((B, HID), jnp.bfloat16))
        o_ref[...] = h_fin.astype(jnp.float32)

    return pl.pallas_call(
        body,
        grid=(nb,),
        in_specs=[
            pl.BlockSpec((T, B, SEQ_DIM), lambda i: (0, i, 0)),
            pl.BlockSpec((SEQ_DIM + HID, 2 * HID), lambda i: (0, 0)),
            pl.BlockSpec((SEQ_DIM, HID), lambda i: (0, 0)),
            pl.BlockSpec((HID, HID), lambda i: (0, 0)),
            pl.BlockSpec((1, 2 * HID), lambda i: (0, 0)),
            pl.BlockSpec((1, HID), lambda i: (0, 0)),
            pl.BlockSpec((1, HID), lambda i: (0, 0)),
        ],
        out_specs=pl.BlockSpec((B, HID), lambda i: (i, 0)),
        out_shape=jax.ShapeDtypeStruct((N, HID), jnp.float32),
    )(x_seq_t, wrz, win, whn, brz, bni, bnh)


def _gru_weights(Wih, Whh, bih, bhh):
    wihT = Wih.T.astype(jnp.bfloat16)
    whhT = Whh.T.astype(jnp.bfloat16)
    wrz = jnp.concatenate([wihT[:, :2 * HID], whhT[:, :2 * HID]], axis=0)
    win = wihT[:, 2 * HID:]
    whn = whhT[:, 2 * HID:]
    brz = (bih[:2 * HID] + bhh[:2 * HID]).reshape(1, -1)
    bni = bih[2 * HID:].reshape(1, -1)
    bnh = bhh[2 * HID:].reshape(1, -1)
    return wrz, win, whn, brz, bni, bnh


def _mat1(mean1_num, deg, x_static, wl1T, bl1, wr1T, wl2T, wr2T, bl2):
    B = 2000
    nb = N // B

    def body(agg_ref, deg_ref, x_ref, wl1_ref, bl1_ref, wr1_ref, wl2_ref,
             wr2_ref, bl2_ref, y2_ref, hr2_ref):
        inv = 1.0 / jnp.maximum(deg_ref[:, 0:1], 1.0)
        mean1 = agg_ref[...] * inv
        h = jnp.maximum(
            jnp.dot(mean1, wl1_ref[...], preferred_element_type=jnp.float32)
            + bl1_ref[...]
            + jnp.dot(x_ref[...], wr1_ref[...],
                      preferred_element_type=jnp.float32), 0.0)
        y2_ref[...] = jnp.dot(h, wl2_ref[...],
                              preferred_element_type=jnp.float32)
        hr2_ref[...] = jnp.dot(h, wr2_ref[...],
                               preferred_element_type=jnp.float32) + bl2_ref[...]

    return pl.pallas_call(
        body,
        grid=(nb,),
        in_specs=[
            pl.BlockSpec((B, STATIC), lambda i: (i, 0)),
            pl.BlockSpec((B, 16), lambda i: (i, 0)),
            pl.BlockSpec((B, STATIC), lambda i: (i, 0)),
            pl.BlockSpec((STATIC, HID), lambda i: (0, 0)),
            pl.BlockSpec((1, HID), lambda i: (0, 0)),
            pl.BlockSpec((STATIC, HID), lambda i: (0, 0)),
            pl.BlockSpec((HID, Z), lambda i: (0, 0)),
            pl.BlockSpec((HID, Z), lambda i: (0, 0)),
            pl.BlockSpec((1, Z), lambda i: (0, 0)),
        ],
        out_specs=[
            pl.BlockSpec((B, Z), lambda i: (i, 0)),
            pl.BlockSpec((B, Z), lambda i: (i, 0)),
        ],
        out_shape=[
            jax.ShapeDtypeStruct((N, Z), jnp.float32),
            jax.ShapeDtypeStruct((N, Z), jnp.float32),
        ],
    )(mean1_num, deg, x_static, wl1T, bl1, wr1T, wl2T, wr2T, bl2)


def _final(agg2, deg, hr2, hf, hb, wfc1T, wfc2T, bfc, wgT, wmT, bfu):
    B = 2000
    nb = N // B

    def body(agg_ref, deg_ref, hr2_ref, hf_ref, hb_ref, wfc1_ref, wfc2_ref,
             bfc_ref, wg_ref, wm_ref, bfu_ref, o_ref):
        inv = 1.0 / jnp.maximum(deg_ref[:, 0:1], 1.0)
        zg = agg_ref[...] * inv + hr2_ref[...]
        zm = (jnp.dot(hf_ref[...], wfc1_ref[...],
                      preferred_element_type=jnp.float32)
              + jnp.dot(hb_ref[...], wfc2_ref[...],
                        preferred_element_type=jnp.float32) + bfc_ref[...])
        o_ref[...] = (jnp.dot(zg, wg_ref[...],
                              preferred_element_type=jnp.float32)
                      + jnp.dot(zm, wm_ref[...],
                                preferred_element_type=jnp.float32)
                      + bfu_ref[...])

    return pl.pallas_call(
        body,
        grid=(nb,),
        in_specs=[
            pl.BlockSpec((B, Z), lambda i: (i, 0)),
            pl.BlockSpec((B, 16), lambda i: (i, 0)),
            pl.BlockSpec((B, Z), lambda i: (i, 0)),
            pl.BlockSpec((B, HID), lambda i: (i, 0)),
            pl.BlockSpec((B, HID), lambda i: (i, 0)),
            pl.BlockSpec((HID, Z), lambda i: (0, 0)),
            pl.BlockSpec((HID, Z), lambda i: (0, 0)),
            pl.BlockSpec((1, Z), lambda i: (0, 0)),
            pl.BlockSpec((Z, Z), lambda i: (0, 0)),
            pl.BlockSpec((Z, Z), lambda i: (0, 0)),
            pl.BlockSpec((1, Z), lambda i: (0, 0)),
        ],
        out_specs=pl.BlockSpec((B, Z), lambda i: (i, 0)),
        out_shape=jax.ShapeDtypeStruct((N, Z), jnp.float32),
    )(agg2, deg, hr2, hf, hb, wfc1T, wfc2T, bfc, wgT, wmT, bfu)


def _to_table(x):
    return x.reshape(N, 4, F).transpose(1, 0, 2).reshape(4 * N, F)


def _from_quarters(a):
    return a.transpose(1, 0, 2).reshape(N, 4 * F)


def kernel(x_static, edge_index, x_seq, Wl1, bl1, Wr1, Wl2, bl2, Wr2,
           Wih_f, Whh_f, bih_f, bhh_f, Wih_b, Whh_b, bih_b, bhh_b,
           Wfc, bfc, Wfu, bfu):
    pad = E_PAD - E
    src = jnp.concatenate([edge_index[0], jnp.zeros((pad,), jnp.int32)])
    dst = jnp.concatenate([edge_index[1], jnp.full((pad,), N, jnp.int32)])
    srcs2 = jnp.stack([src, src + 2 * N])
    zeros_f = jnp.zeros((RPT, F), jnp.float32)
    zeros_d = jnp.zeros((RPT, 16), jnp.float32)
    ones_c = jnp.ones((CH, 16), jnp.float32)

    agg1, deg = _sc_agg(_to_table(x_static), srcs2, dst,
                        zeros_f, zeros_d, ones_c, True)

    x_seq_t = jnp.swapaxes(x_seq, 0, 1).astype(jnp.bfloat16)
    hf = _gru_dir(x_seq_t, *_gru_weights(Wih_f, Whh_f, bih_f, bhh_f), False)

    y2, hr2 = _mat1(_from_quarters(agg1), deg, x_static, Wl1.T,
                    bl1.reshape(1, -1), Wr1.T, Wl2.T, Wr2.T,
                    bl2.reshape(1, -1))

    (agg2,) = _sc_agg(_to_table(y2), srcs2, dst,
                      zeros_f, zeros_d, ones_c, False)

    hb = _gru_dir(x_seq_t, *_gru_weights(Wih_b, Whh_b, bih_b, bhh_b), True)

    return _final(_from_quarters(agg2), deg, hr2, hf, hb,
                  Wfc[:, :HID].T, Wfc[:, HID:].T, bfc.reshape(1, -1),
                  Wfu[:, :Z].T, Wfu[:, Z:].T, bfu.reshape(1, -1))


# speedup vs baseline: 1.2502x; 1.0986x over previous
import functools

import jax
import jax.numpy as jnp
from jax import lax
from jax.experimental import pallas as pl
from jax.experimental.pallas import tpu as pltpu
from jax.experimental.pallas import tpu_sc as plsc

N = 10000
E = 160000
STATIC = 256
SEQ_DIM = 16
T = 50
HID = 512
Z = 256
F = 64

NSUB = 16
E_PAD = 163840
EPT = E_PAD // NSUB
CH = 512
NCHUNK = EPT // CH
NROW = 10112
RPT = NROW // NSUB
LAST_L = N - (NSUB - 1) * RPT


def _sc_agg(table, srcs2, dst, zeros_f, zeros_d, ones_c, with_deg):
    mesh = plsc.VectorSubcoreMesh(core_axis_name="c", subcore_axis_name="s")
    out_type = [jax.ShapeDtypeStruct((4, N, F), jnp.float32)]
    scratch = [
        pltpu.VMEM((CH, F), jnp.float32),
        pltpu.VMEM((CH, F), jnp.float32),
        pltpu.VMEM((CH,), jnp.int32),
        pltpu.VMEM((CH,), jnp.int32),
        pltpu.VMEM((CH,), jnp.int32),
        pltpu.VMEM((CH,), jnp.int32),
        pltpu.SemaphoreType.DMA,
        pltpu.SemaphoreType.DMA,
        pltpu.VMEM_SHARED((NROW, F), jnp.float32),
    ]
    if with_deg:
        out_type.append(jax.ShapeDtypeStruct((N, 16), jnp.float32))
        scratch += [
            pltpu.VMEM((CH, 16), jnp.float32),
            pltpu.VMEM_SHARED((NROW, 16), jnp.float32),
        ]

    def body(table_h, srcs_h, dst_h, zf_h, zd_h, ones_h, agg_h, *rest):
        if with_deg:
            (deg_h, rows_a, rows_b, gidx_a, gidx_b, didx_a, didx_b,
             sem_a, sem_b, acc_sh, ones_v, deg_sh) = rest
        else:
            (rows_a, rows_b, gidx_a, gidx_b, didx_a, didx_b,
             sem_a, sem_b, acc_sh) = rest
        c = lax.axis_index("c")
        s = lax.axis_index("s")
        row0 = pl.multiple_of(s * RPT, 8)

        def zero_acc():
            pltpu.sync_copy(zf_h, acc_sh.at[pl.ds(row0, RPT)])

        def edge_pass(p):
            def load_idx(k, gidx_v, didx_v):
                base = pl.multiple_of(s * EPT + k * CH, 128)
                pltpu.sync_copy(srcs_h.at[c].at[pl.ds(base, CH)], gidx_v)
                pltpu.sync_copy(dst_h.at[pl.ds(base, CH)], didx_v)
                if p:
                    @pl.loop(0, CH // 16)
                    def _adj(i):
                        sl = pl.ds(i * 16, 16)
                        gidx_v[sl] = gidx_v[sl] + jnp.int32(p * N)

            def start_gather(gidx_v, rows_v, sem):
                pltpu.async_copy(table_h.at[gidx_v], rows_v, sem)

            def finish(rows_v, gidx_v, didx_v, sem):
                pltpu.make_async_copy(table_h.at[gidx_v], rows_v, sem).wait()
                pltpu.sync_copy(rows_v, acc_sh.at[didx_v], add=True)
                if with_deg and p == 0:
                    pltpu.sync_copy(ones_v, deg_sh.at[didx_v], add=True)

            load_idx(0, gidx_a, didx_a)
            start_gather(gidx_a, rows_a, sem_a)

            @pl.loop(0, NCHUNK, step=2)
            def _edges(k):
                load_idx(k + 1, gidx_b, didx_b)
                start_gather(gidx_b, rows_b, sem_b)
                finish(rows_a, gidx_a, didx_a, sem_a)

                @pl.when(k + 2 < NCHUNK)
                def _():
                    load_idx(k + 2, gidx_a, didx_a)
                    start_gather(gidx_a, rows_a, sem_a)

                finish(rows_b, gidx_b, didx_b, sem_b)

        def writeback(p, length):
            pltpu.sync_copy(acc_sh.at[pl.ds(row0, length)],
                            agg_h.at[2 * c + p].at[pl.ds(row0, length)])
            if with_deg and p == 0:
                @pl.when(c == 0)
                def _():
                    pltpu.sync_copy(deg_sh.at[pl.ds(row0, length)],
                                    deg_h.at[pl.ds(row0, length)])

        def writeback_sized(p):
            @pl.when(s < NSUB - 1)
            def _():
                writeback(p, RPT)

            @pl.when(s == NSUB - 1)
            def _():
                writeback(p, LAST_L)

        zero_acc()
        if with_deg:
            pltpu.sync_copy(zd_h, deg_sh.at[pl.ds(row0, RPT)])
            pltpu.sync_copy(ones_h, ones_v)
        plsc.subcore_barrier()
        edge_pass(0)
        plsc.subcore_barrier()
        writeback_sized(0)
        zero_acc()
        plsc.subcore_barrier()
        edge_pass(1)
        plsc.subcore_barrier()
        writeback_sized(1)

    k = pl.kernel(body, mesh=mesh, out_type=out_type, scratch_types=scratch,
                  compiler_params=pltpu.CompilerParams(
                      use_tc_tiling_on_sc=False))
    return k(table, srcs2, dst, zeros_f, zeros_d, ones_c)


def _gru_dir(x_seq_t, wrz, win, whn, brz, bni, bnh, reverse):
    B = 1000
    nb = N // B

    def body(x_ref, wrz_ref, win_ref, whn_ref, brz_ref, bni_ref, bnh_ref,
             o_ref):
        def step(t, h):
            tt = (T - 1 - t) if reverse else t
            xt = x_ref[tt]
            xh = jnp.concatenate([xt, h], axis=1)
            rz = jax.nn.sigmoid(
                jnp.dot(xh, wrz_ref[...],
                        preferred_element_type=jnp.float32)
                + brz_ref[...]).astype(jnp.bfloat16)
            gn = (jnp.dot(xt, win_ref[...],
                          preferred_element_type=jnp.float32)
                  + bni_ref[...]).astype(jnp.bfloat16)
            gh = (jnp.dot(h, whn_ref[...],
                          preferred_element_type=jnp.float32)
                  + bnh_ref[...]).astype(jnp.bfloat16)
            n = jnp.tanh(gn + rz[:, :HID] * gh)
            z = rz[:, HID:]
            return (1.0 - z) * n + z * h

        h_fin = lax.fori_loop(0, T, step, jnp.zeros((B, HID), jnp.bfloat16))
        o_ref[...] = h_fin.astype(jnp.float32)

    return pl.pallas_call(
        body,
        grid=(nb,),
        in_specs=[
            pl.BlockSpec((T, B, SEQ_DIM), lambda i: (0, i, 0)),
            pl.BlockSpec((SEQ_DIM + HID, 2 * HID), lambda i: (0, 0)),
            pl.BlockSpec((SEQ_DIM, HID), lambda i: (0, 0)),
            pl.BlockSpec((HID, HID), lambda i: (0, 0)),
            pl.BlockSpec((1, 2 * HID), lambda i: (0, 0)),
            pl.BlockSpec((1, HID), lambda i: (0, 0)),
            pl.BlockSpec((1, HID), lambda i: (0, 0)),
        ],
        out_specs=pl.BlockSpec((B, HID), lambda i: (i, 0)),
        out_shape=jax.ShapeDtypeStruct((N, HID), jnp.float32),
    )(x_seq_t, wrz, win, whn, brz, bni, bnh)


def _gru_weights(Wih, Whh, bih, bhh):
    wihT = Wih.T.astype(jnp.bfloat16)
    whhT = Whh.T.astype(jnp.bfloat16)
    wrz = jnp.concatenate([wihT[:, :2 * HID], whhT[:, :2 * HID]], axis=0)
    win = wihT[:, 2 * HID:]
    whn = whhT[:, 2 * HID:]
    brz = (bih[:2 * HID] + bhh[:2 * HID]).reshape(1, -1)
    bni = bih[2 * HID:].reshape(1, -1)
    bnh = bhh[2 * HID:].reshape(1, -1)
    return wrz, win, whn, brz, bni, bnh


def _mat1(mean1_num, deg, x_static, wl1T, bl1, wr1T, wl2T, wr2T, bl2):
    B = 2000
    nb = N // B

    def body(agg_ref, deg_ref, x_ref, wl1_ref, bl1_ref, wr1_ref, wl2_ref,
             wr2_ref, bl2_ref, y2_ref, hr2_ref):
        inv = 1.0 / jnp.maximum(deg_ref[:, 0:1], 1.0)
        mean1 = jnp.concatenate(
            [agg_ref[q] for q in range(4)], axis=1) * inv
        h = jnp.maximum(
            jnp.dot(mean1, wl1_ref[...], preferred_element_type=jnp.float32)
            + bl1_ref[...]
            + jnp.dot(x_ref[...], wr1_ref[...],
                      preferred_element_type=jnp.float32), 0.0)
        y2 = jnp.dot(h, wl2_ref[...], preferred_element_type=jnp.float32)
        for q in range(4):
            y2_ref[q] = y2[:, q * F:(q + 1) * F]
        hr2_ref[...] = jnp.dot(h, wr2_ref[...],
                               preferred_element_type=jnp.float32) + bl2_ref[...]

    return pl.pallas_call(
        body,
        grid=(nb,),
        in_specs=[
            pl.BlockSpec((4, B, F), lambda i: (0, i, 0)),
            pl.BlockSpec((B, 16), lambda i: (i, 0)),
            pl.BlockSpec((B, STATIC), lambda i: (i, 0)),
            pl.BlockSpec((STATIC, HID), lambda i: (0, 0)),
            pl.BlockSpec((1, HID), lambda i: (0, 0)),
            pl.BlockSpec((STATIC, HID), lambda i: (0, 0)),
            pl.BlockSpec((HID, Z), lambda i: (0, 0)),
            pl.BlockSpec((HID, Z), lambda i: (0, 0)),
            pl.BlockSpec((1, Z), lambda i: (0, 0)),
        ],
        out_specs=[
            pl.BlockSpec((4, B, F), lambda i: (0, i, 0)),
            pl.BlockSpec((B, Z), lambda i: (i, 0)),
        ],
        out_shape=[
            jax.ShapeDtypeStruct((4, N, F), jnp.float32),
            jax.ShapeDtypeStruct((N, Z), jnp.float32),
        ],
    )(mean1_num, deg, x_static, wl1T, bl1, wr1T, wl2T, wr2T, bl2)


def _final(agg2, deg, hr2, hf, hb, wfc1T, wfc2T, bfc, wgT, wmT, bfu):
    B = 2000
    nb = N // B

    def body(agg_ref, deg_ref, hr2_ref, hf_ref, hb_ref, wfc1_ref, wfc2_ref,
             bfc_ref, wg_ref, wm_ref, bfu_ref, o_ref):
        inv = 1.0 / jnp.maximum(deg_ref[:, 0:1], 1.0)
        zg = (jnp.concatenate([agg_ref[q] for q in range(4)], axis=1) * inv
              + hr2_ref[...])
        zm = (jnp.dot(hf_ref[...], wfc1_ref[...],
                      preferred_element_type=jnp.float32)
              + jnp.dot(hb_ref[...], wfc2_ref[...],
                        preferred_element_type=jnp.float32) + bfc_ref[...])
        o_ref[...] = (jnp.dot(zg, wg_ref[...],
                              preferred_element_type=jnp.float32)
                      + jnp.dot(zm, wm_ref[...],
                                preferred_element_type=jnp.float32)
                      + bfu_ref[...])

    return pl.pallas_call(
        body,
        grid=(nb,),
        in_specs=[
            pl.BlockSpec((4, B, F), lambda i: (0, i, 0)),
            pl.BlockSpec((B, 16), lambda i: (i, 0)),
            pl.BlockSpec((B, Z), lambda i: (i, 0)),
            pl.BlockSpec((B, HID), lambda i: (i, 0)),
            pl.BlockSpec((B, HID), lambda i: (i, 0)),
            pl.BlockSpec((HID, Z), lambda i: (0, 0)),
            pl.BlockSpec((HID, Z), lambda i: (0, 0)),
            pl.BlockSpec((1, Z), lambda i: (0, 0)),
            pl.BlockSpec((Z, Z), lambda i: (0, 0)),
            pl.BlockSpec((Z, Z), lambda i: (0, 0)),
            pl.BlockSpec((1, Z), lambda i: (0, 0)),
        ],
        out_specs=pl.BlockSpec((B, Z), lambda i: (i, 0)),
        out_shape=jax.ShapeDtypeStruct((N, Z), jnp.float32),
    )(agg2, deg, hr2, hf, hb, wfc1T, wfc2T, bfc, wgT, wmT, bfu)


def _to_table(x):
    return x.reshape(N, 4, F).transpose(1, 0, 2).reshape(4 * N, F)


def kernel(x_static, edge_index, x_seq, Wl1, bl1, Wr1, Wl2, bl2, Wr2,
           Wih_f, Whh_f, bih_f, bhh_f, Wih_b, Whh_b, bih_b, bhh_b,
           Wfc, bfc, Wfu, bfu):
    pad = E_PAD - E
    src = jnp.concatenate([edge_index[0], jnp.zeros((pad,), jnp.int32)])
    dst = jnp.concatenate([edge_index[1], jnp.full((pad,), N, jnp.int32)])
    srcs2 = jnp.stack([src, src + 2 * N])
    zeros_f = jnp.zeros((RPT, F), jnp.float32)
    zeros_d = jnp.zeros((RPT, 16), jnp.float32)
    ones_c = jnp.ones((CH, 16), jnp.float32)

    agg1, deg = _sc_agg(_to_table(x_static), srcs2, dst,
                        zeros_f, zeros_d, ones_c, True)

    x_seq_t = jnp.swapaxes(x_seq, 0, 1).astype(jnp.bfloat16)
    hf = _gru_dir(x_seq_t, *_gru_weights(Wih_f, Whh_f, bih_f, bhh_f), False)

    y2q, hr2 = _mat1(agg1, deg, x_static, Wl1.T,
                     bl1.reshape(1, -1), Wr1.T, Wl2.T, Wr2.T,
                     bl2.reshape(1, -1))

    (agg2,) = _sc_agg(y2q.reshape(4 * N, F), srcs2, dst,
                      zeros_f, zeros_d, ones_c, False)

    hb = _gru_dir(x_seq_t, *_gru_weights(Wih_b, Whh_b, bih_b, bhh_b), True)

    return _final(agg2, deg, hr2, hf, hb,
                  Wfc[:, :HID].T, Wfc[:, HID:].T, bfc.reshape(1, -1),
                  Wfu[:, :Z].T, Wfu[:, Z:].T, bfu.reshape(1, -1))


# speedup vs baseline: 1.2632x; 1.0104x over previous
import functools

import jax
import jax.numpy as jnp
from jax import lax
from jax.experimental import pallas as pl
from jax.experimental.pallas import tpu as pltpu
from jax.experimental.pallas import tpu_sc as plsc

N = 10000
E = 160000
STATIC = 256
SEQ_DIM = 16
T = 50
HID = 512
Z = 256
F = 64

NSUB = 16
E_PAD = 163840
EPT = E_PAD // NSUB
CH = 512
NCHUNK = EPT // CH
NROW = 10112
RPT = NROW // NSUB
LAST_L = N - (NSUB - 1) * RPT


def _sc_agg(table, srcs2, dst, zeros_f, zeros_d, ones_c, with_deg):
    mesh = plsc.VectorSubcoreMesh(core_axis_name="c", subcore_axis_name="s")
    out_type = [jax.ShapeDtypeStruct((4, N, F), jnp.float32)]
    scratch = [
        pltpu.VMEM((CH, F), jnp.float32),
        pltpu.VMEM((CH, F), jnp.float32),
        pltpu.VMEM((CH,), jnp.int32),
        pltpu.VMEM((CH,), jnp.int32),
        pltpu.VMEM((CH,), jnp.int32),
        pltpu.VMEM((CH,), jnp.int32),
        pltpu.SemaphoreType.DMA,
        pltpu.SemaphoreType.DMA,
        pltpu.VMEM_SHARED((NROW, F), jnp.float32),
    ]
    if with_deg:
        out_type.append(jax.ShapeDtypeStruct((N, 16), jnp.float32))
        scratch += [
            pltpu.VMEM((CH, 16), jnp.float32),
            pltpu.VMEM_SHARED((NROW, 16), jnp.float32),
        ]

    def body(table_h, srcs_h, dst_h, zf_h, zd_h, ones_h, agg_h, *rest):
        if with_deg:
            (deg_h, rows_a, rows_b, gidx_a, gidx_b, didx_a, didx_b,
             sem_a, sem_b, acc_sh, ones_v, deg_sh) = rest
        else:
            (rows_a, rows_b, gidx_a, gidx_b, didx_a, didx_b,
             sem_a, sem_b, acc_sh) = rest
        c = lax.axis_index("c")
        s = lax.axis_index("s")
        row0 = pl.multiple_of(s * RPT, 8)

        def zero_acc():
            pltpu.sync_copy(zf_h, acc_sh.at[pl.ds(row0, RPT)])

        def edge_pass(p):
            def load_idx(k, gidx_v, didx_v):
                base = pl.multiple_of(s * EPT + k * CH, 128)
                pltpu.sync_copy(srcs_h.at[c].at[pl.ds(base, CH)], gidx_v)
                pltpu.sync_copy(dst_h.at[pl.ds(base, CH)], didx_v)
                if p:
                    @pl.loop(0, CH // 16)
                    def _adj(i):
                        sl = pl.ds(i * 16, 16)
                        gidx_v[sl] = gidx_v[sl] + jnp.int32(p * N)

            def start_gather(gidx_v, rows_v, sem):
                pltpu.async_copy(table_h.at[gidx_v], rows_v, sem)

            def finish(rows_v, gidx_v, didx_v, sem):
                pltpu.make_async_copy(table_h.at[gidx_v], rows_v, sem).wait()
                pltpu.sync_copy(rows_v, acc_sh.at[didx_v], add=True)
                if with_deg and p == 0:
                    pltpu.sync_copy(ones_v, deg_sh.at[didx_v], add=True)

            load_idx(0, gidx_a, didx_a)
            start_gather(gidx_a, rows_a, sem_a)

            @pl.loop(0, NCHUNK, step=2)
            def _edges(k):
                load_idx(k + 1, gidx_b, didx_b)
                start_gather(gidx_b, rows_b, sem_b)
                finish(rows_a, gidx_a, didx_a, sem_a)

                @pl.when(k + 2 < NCHUNK)
                def _():
                    load_idx(k + 2, gidx_a, didx_a)
                    start_gather(gidx_a, rows_a, sem_a)

                finish(rows_b, gidx_b, didx_b, sem_b)

        def writeback(p, length):
            pltpu.sync_copy(acc_sh.at[pl.ds(row0, length)],
                            agg_h.at[2 * c + p].at[pl.ds(row0, length)])
            if with_deg and p == 0:
                @pl.when(c == 0)
                def _():
                    pltpu.sync_copy(deg_sh.at[pl.ds(row0, length)],
                                    deg_h.at[pl.ds(row0, length)])

        def writeback_sized(p):
            @pl.when(s < NSUB - 1)
            def _():
                writeback(p, RPT)

            @pl.when(s == NSUB - 1)
            def _():
                writeback(p, LAST_L)

        zero_acc()
        if with_deg:
            pltpu.sync_copy(zd_h, deg_sh.at[pl.ds(row0, RPT)])
            pltpu.sync_copy(ones_h, ones_v)
        plsc.subcore_barrier()
        edge_pass(0)
        plsc.subcore_barrier()
        writeback_sized(0)
        zero_acc()
        plsc.subcore_barrier()
        edge_pass(1)
        plsc.subcore_barrier()
        writeback_sized(1)

    k = pl.kernel(body, mesh=mesh, out_type=out_type, scratch_types=scratch,
                  compiler_params=pltpu.CompilerParams(
                      use_tc_tiling_on_sc=False))
    return k(table, srcs2, dst, zeros_f, zeros_d, ones_c)


def _gru_dir(x_seq_t, wrz, win, whn, brz, bni, bnh, reverse):
    B = 1000
    nb = N // B

    def body(x_ref, wrz_ref, win_ref, whn_ref, brz_ref, bni_ref, bnh_ref,
             o_ref):
        def step(t, h):
            tt = (T - 1 - t) if reverse else t
            xt = x_ref[tt]
            xh = jnp.concatenate([xt, h], axis=1)
            rz = jax.nn.sigmoid(
                jnp.dot(xh, wrz_ref[...],
                        preferred_element_type=jnp.float32)
                + brz_ref[...]).astype(jnp.bfloat16)
            gn = (jnp.dot(xt, win_ref[...],
                          preferred_element_type=jnp.float32)
                  + bni_ref[...]).astype(jnp.bfloat16)
            gh = (jnp.dot(h, whn_ref[...],
                          preferred_element_type=jnp.float32)
                  + bnh_ref[...]).astype(jnp.bfloat16)
            n = jnp.tanh(gn + rz[:, :HID] * gh)
            z = rz[:, HID:]
            return (1.0 - z) * n + z * h

        h_fin = lax.fori_loop(0, T, step, jnp.zeros((B, HID), jnp.bfloat16))
        o_ref[...] = h_fin.astype(jnp.float32)

    return pl.pallas_call(
        body,
        grid=(nb,),
        in_specs=[
            pl.BlockSpec((T, B, SEQ_DIM), lambda i: (0, i, 0)),
            pl.BlockSpec((SEQ_DIM + HID, 2 * HID), lambda i: (0, 0)),
            pl.BlockSpec((SEQ_DIM, HID), lambda i: (0, 0)),
            pl.BlockSpec((HID, HID), lambda i: (0, 0)),
            pl.BlockSpec((1, 2 * HID), lambda i: (0, 0)),
            pl.BlockSpec((1, HID), lambda i: (0, 0)),
            pl.BlockSpec((1, HID), lambda i: (0, 0)),
        ],
        out_specs=pl.BlockSpec((B, HID), lambda i: (i, 0)),
        out_shape=jax.ShapeDtypeStruct((N, HID), jnp.float32),
        compiler_params=pltpu.CompilerParams(
            dimension_semantics=("parallel",)),
    )(x_seq_t, wrz, win, whn, brz, bni, bnh)


def _gru_weights(Wih, Whh, bih, bhh):
    wihT = Wih.T.astype(jnp.bfloat16)
    whhT = Whh.T.astype(jnp.bfloat16)
    wrz = jnp.concatenate([wihT[:, :2 * HID], whhT[:, :2 * HID]], axis=0)
    win = wihT[:, 2 * HID:]
    whn = whhT[:, 2 * HID:]
    brz = (bih[:2 * HID] + bhh[:2 * HID]).reshape(1, -1)
    bni = bih[2 * HID:].reshape(1, -1)
    bnh = bhh[2 * HID:].reshape(1, -1)
    return wrz, win, whn, brz, bni, bnh


def _mat1(mean1_num, deg, x_static, wl1T, bl1, wr1T, wl2T, wr2T, bl2):
    B = 2000
    nb = N // B

    def body(agg_ref, deg_ref, x_ref, wl1_ref, bl1_ref, wr1_ref, wl2_ref,
             wr2_ref, bl2_ref, y2_ref, hr2_ref):
        inv = 1.0 / jnp.maximum(deg_ref[:, 0:1], 1.0)
        mean1 = jnp.concatenate(
            [agg_ref[q] for q in range(4)], axis=1) * inv
        h = jnp.maximum(
            jnp.dot(mean1, wl1_ref[...], preferred_element_type=jnp.float32)
            + bl1_ref[...]
            + jnp.dot(x_ref[...], wr1_ref[...],
                      preferred_element_type=jnp.float32), 0.0)
        y2 = jnp.dot(h, wl2_ref[...], preferred_element_type=jnp.float32)
        for q in range(4):
            y2_ref[q] = y2[:, q * F:(q + 1) * F]
        hr2_ref[...] = jnp.dot(h, wr2_ref[...],
                               preferred_element_type=jnp.float32) + bl2_ref[...]

    return pl.pallas_call(
        body,
        grid=(nb,),
        in_specs=[
            pl.BlockSpec((4, B, F), lambda i: (0, i, 0)),
            pl.BlockSpec((B, 16), lambda i: (i, 0)),
            pl.BlockSpec((B, STATIC), lambda i: (i, 0)),
            pl.BlockSpec((STATIC, HID), lambda i: (0, 0)),
            pl.BlockSpec((1, HID), lambda i: (0, 0)),
            pl.BlockSpec((STATIC, HID), lambda i: (0, 0)),
            pl.BlockSpec((HID, Z), lambda i: (0, 0)),
            pl.BlockSpec((HID, Z), lambda i: (0, 0)),
            pl.BlockSpec((1, Z), lambda i: (0, 0)),
        ],
        out_specs=[
            pl.BlockSpec((4, B, F), lambda i: (0, i, 0)),
            pl.BlockSpec((B, Z), lambda i: (i, 0)),
        ],
        out_shape=[
            jax.ShapeDtypeStruct((4, N, F), jnp.float32),
            jax.ShapeDtypeStruct((N, Z), jnp.float32),
        ],
        compiler_params=pltpu.CompilerParams(
            dimension_semantics=("parallel",)),
    )(mean1_num, deg, x_static, wl1T, bl1, wr1T, wl2T, wr2T, bl2)


def _final(agg2, deg, hr2, hf, hb, wfc1T, wfc2T, bfc, wgT, wmT, bfu):
    B = 2000
    nb = N // B

    def body(agg_ref, deg_ref, hr2_ref, hf_ref, hb_ref, wfc1_ref, wfc2_ref,
             bfc_ref, wg_ref, wm_ref, bfu_ref, o_ref):
        inv = 1.0 / jnp.maximum(deg_ref[:, 0:1], 1.0)
        zg = (jnp.concatenate([agg_ref[q] for q in range(4)], axis=1) * inv
              + hr2_ref[...])
        zm = (jnp.dot(hf_ref[...], wfc1_ref[...],
                      preferred_element_type=jnp.float32)
              + jnp.dot(hb_ref[...], wfc2_ref[...],
                        preferred_element_type=jnp.float32) + bfc_ref[...])
        o_ref[...] = (jnp.dot(zg, wg_ref[...],
                              preferred_element_type=jnp.float32)
                      + jnp.dot(zm, wm_ref[...],
                                preferred_element_type=jnp.float32)
                      + bfu_ref[...])

    return pl.pallas_call(
        body,
        grid=(nb,),
        in_specs=[
            pl.BlockSpec((4, B, F), lambda i: (0, i, 0)),
            pl.BlockSpec((B, 16), lambda i: (i, 0)),
            pl.BlockSpec((B, Z), lambda i: (i, 0)),
            pl.BlockSpec((B, HID), lambda i: (i, 0)),
            pl.BlockSpec((B, HID), lambda i: (i, 0)),
            pl.BlockSpec((HID, Z), lambda i: (0, 0)),
            pl.BlockSpec((HID, Z), lambda i: (0, 0)),
            pl.BlockSpec((1, Z), lambda i: (0, 0)),
            pl.BlockSpec((Z, Z), lambda i: (0, 0)),
            pl.BlockSpec((Z, Z), lambda i: (0, 0)),
            pl.BlockSpec((1, Z), lambda i: (0, 0)),
        ],
        out_specs=pl.BlockSpec((B, Z), lambda i: (i, 0)),
        out_shape=jax.ShapeDtypeStruct((N, Z), jnp.float32),
        compiler_params=pltpu.CompilerParams(
            dimension_semantics=("parallel",)),
    )(agg2, deg, hr2, hf, hb, wfc1T, wfc2T, bfc, wgT, wmT, bfu)


def _to_table(x):
    return x.reshape(N, 4, F).transpose(1, 0, 2).reshape(4 * N, F)


def kernel(x_static, edge_index, x_seq, Wl1, bl1, Wr1, Wl2, bl2, Wr2,
           Wih_f, Whh_f, bih_f, bhh_f, Wih_b, Whh_b, bih_b, bhh_b,
           Wfc, bfc, Wfu, bfu):
    pad = E_PAD - E
    src = jnp.concatenate([edge_index[0], jnp.zeros((pad,), jnp.int32)])
    dst = jnp.concatenate([edge_index[1], jnp.full((pad,), N, jnp.int32)])
    srcs2 = jnp.stack([src, src + 2 * N])
    zeros_f = jnp.zeros((RPT, F), jnp.float32)
    zeros_d = jnp.zeros((RPT, 16), jnp.float32)
    ones_c = jnp.ones((CH, 16), jnp.float32)

    agg1, deg = _sc_agg(_to_table(x_static), srcs2, dst,
                        zeros_f, zeros_d, ones_c, True)

    x_seq_t = jnp.swapaxes(x_seq, 0, 1).astype(jnp.bfloat16)
    hf = _gru_dir(x_seq_t, *_gru_weights(Wih_f, Whh_f, bih_f, bhh_f), False)

    y2q, hr2 = _mat1(agg1, deg, x_static, Wl1.T,
                     bl1.reshape(1, -1), Wr1.T, Wl2.T, Wr2.T,
                     bl2.reshape(1, -1))

    (agg2,) = _sc_agg(y2q.reshape(4 * N, F), srcs2, dst,
                      zeros_f, zeros_d, ones_c, False)

    hb = _gru_dir(x_seq_t, *_gru_weights(Wih_b, Whh_b, bih_b, bhh_b), True)

    return _final(agg2, deg, hr2, hf, hb,
                  Wfc[:, :HID].T, Wfc[:, HID:].T, bfc.reshape(1, -1),
                  Wfu[:, :Z].T, Wfu[:, Z:].T, bfu.reshape(1, -1))
